# baseline (speedup 1.0000x reference)
import functools

import jax
import jax.numpy as jnp
from jax import lax
from jax.experimental import pallas as pl
from jax.experimental.pallas import tpu as pltpu
from jax.experimental.pallas import tpu_sc as plsc

B, T, D = 16, 576, 64
TOKENS = B * T
K = 8192
TBLK = 512
KBLK = 2048
NC, NS = 2, 16
NW = NC * NS
BPW = TOKENS // NW
GCH = 96
NCH = BPW // GCH
BIG_I32 = 2**30


def _distances_block(x, w_ref, c):
    wc = w_ref[pl.ds(c * KBLK, KBLK), :]
    w2 = jnp.sum(wc * wc, axis=1, keepdims=True)
    x2 = jnp.sum(x * x, axis=1, keepdims=True)
    s = lax.dot_general(
        x.astype(jnp.bfloat16), wc.astype(jnp.bfloat16),
        (((1,), (1,)), ((), ())), preferred_element_type=jnp.float32)
    d2 = (x2 + w2.reshape(1, KBLK)) - 2.0 * s
    return jnp.sqrt(jnp.maximum(d2, 0.0))


def _argmin_over_codebook(x, w_ref, idx_ref, half_ref):
    run_min = jnp.full((TBLK, 1), jnp.inf, dtype=jnp.float32)
    run_idx = jnp.zeros((TBLK, 1), dtype=jnp.int32)
    for c in range(K // KBLK):
        ds = _distances_block(x, w_ref, c)
        m = jnp.min(ds, axis=1, keepdims=True)
        iota = lax.broadcasted_iota(jnp.int32, (TBLK, KBLK), 1) + c * KBLK
        i_local = jnp.min(jnp.where(ds == m, iota, BIG_I32), axis=1,
                          keepdims=True)
        better = m < run_min
        run_min = jnp.where(better, m, run_min)
        run_idx = jnp.where(better, i_local, run_idx)
    idx_ref[...] = run_idx
    half_ref[...] = run_idx >> 1


def _argmin_body1(x_ref, w_ref, idx_ref, half_ref):
    _argmin_over_codebook(x_ref[...], w_ref, idx_ref, half_ref)


def _argmin_body2(x_ref, qp_ref, i_ref, w_ref, idx_ref, half_ref):
    q_prev = _select_half(qp_ref[...], i_ref[...])
    _argmin_over_codebook(x_ref[...] - q_prev, w_ref, idx_ref, half_ref)


def _select_half(qp, idx):
    even = (idx & 1) == 0
    return jnp.where(even, qp[:, :D], qp[:, D:])


def _tc_argmin(stage2):
    body = _argmin_body2 if stage2 else _argmin_body1
    tok_spec = pl.BlockSpec((TBLK, D), lambda i: (i, 0))
    qp_spec = pl.BlockSpec((TBLK, 2 * D), lambda i: (i, 0))
    i_spec = pl.BlockSpec((TBLK, 1), lambda i: (i, 0))
    in_specs = [tok_spec] + ([qp_spec, i_spec] if stage2 else []) + [
        pl.BlockSpec((K, D), lambda i: (0, 0))]
    return pl.pallas_call(
        body,
        grid=(TOKENS // TBLK,),
        in_specs=in_specs,
        out_specs=[i_spec, i_spec],
        out_shape=[jax.ShapeDtypeStruct((TOKENS, 1), jnp.int32),
                   jax.ShapeDtypeStruct((TOKENS, 1), jnp.int32)],
    )


_argmin_stage1 = _tc_argmin(stage2=False)
_argmin_stage2 = _tc_argmin(stage2=True)


def _sc_gather_body(tbl_ref, idx_ref, out_ref, idx_v, r0, r1, r2, sem):
    wid = lax.axis_index("s") * NC + lax.axis_index("c")
    pltpu.sync_copy(idx_ref.at[wid], idx_v)
    bufs = (r0, r1, r2)
    copies = [
        pltpu.async_copy(tbl_ref.at[idx_v.at[j]], bufs[j], sem)
        for j in range(NCH)
    ]
    for cp in copies:
        cp.wait()
    for j in range(NCH):
        pltpu.sync_copy(bufs[j], out_ref.at[pl.ds(wid * BPW + j * GCH, GCH)])


@functools.cache
def _sc_gather():
    return pl.kernel(
        _sc_gather_body,
        mesh=plsc.VectorSubcoreMesh(core_axis_name="c", subcore_axis_name="s"),
        out_type=jax.ShapeDtypeStruct((TOKENS, 2 * D), jnp.float32),
        scratch_types=[
            pltpu.VMEM((NCH, GCH), jnp.int32),
            pltpu.VMEM((GCH, 2 * D), jnp.float32),
            pltpu.VMEM((GCH, 2 * D), jnp.float32),
            pltpu.VMEM((GCH, 2 * D), jnp.float32),
            pltpu.SemaphoreType.DMA,
        ],
    )


def _final_body(x_ref, q0p_ref, i0_ref, q1p_ref, i1_ref, loss_ref, out_ref):
    x = x_ref[...]
    q0 = _select_half(q0p_ref[...], i0_ref[...])
    q1 = _select_half(q1p_ref[...], i1_ref[...])
    r1 = x - q0
    r2 = r1 - q1
    code_sum = q0 + q1
    out_ref[...] = x + (code_sum - x)
    n = float(TOKENS * D)
    l1 = jnp.sum(r1 * r1) / n
    l2 = jnp.sum(r2 * r2) / n
    loss_ref[0, 0] = ((l1 + 0.25 * l1) + l2) + 0.25 * l2


_final = pl.pallas_call(
    _final_body,
    in_specs=[
        pl.BlockSpec((TOKENS, D), lambda: (0, 0)),
        pl.BlockSpec((TOKENS, 2 * D), lambda: (0, 0)),
        pl.BlockSpec((TOKENS, 1), lambda: (0, 0)),
        pl.BlockSpec((TOKENS, 2 * D), lambda: (0, 0)),
        pl.BlockSpec((TOKENS, 1), lambda: (0, 0)),
    ],
    out_specs=[
        pl.BlockSpec(memory_space=pltpu.SMEM),
        pl.BlockSpec((TOKENS, D), lambda: (0, 0)),
    ],
    out_shape=[
        jax.ShapeDtypeStruct((1, 1), jnp.float32),
        jax.ShapeDtypeStruct((TOKENS, D), jnp.float32),
    ],
)


def kernel(latent, W0, W1):
    x = latent.reshape(TOKENS, D)
    gather = _sc_gather()
    W0p = W0.reshape(K // 2, 2 * D)
    W1p = W1.reshape(K // 2, 2 * D)
    idx0, half0 = _argmin_stage1(x, W0)
    q0p = gather(W0p, half0.reshape(NW, NCH, GCH))
    idx1, half1 = _argmin_stage2(x, q0p, idx0, W1)
    q1p = gather(W1p, half1.reshape(NW, NCH, GCH))
    loss, out = _final(x, q0p, idx0, q1p, idx1)
    return loss[0, 0], out.reshape(B, T, D)
